# baseline (speedup 1.0000x reference)
import functools

import jax
import jax.numpy as jnp
from jax import lax
from jax.experimental import pallas as pl
from jax.experimental.pallas import tpu as pltpu
from jax.experimental.pallas import tpu_sc as plsc

NC = 2
NS = 16
NW = NC * NS


def _seg_sum(N, Npad, K, CHUNK, D):
    NZT = Npad // NS
    NTT = N // NS
    mesh = plsc.VectorSubcoreMesh(core_axis_name="c", subcore_axis_name="s")

    @functools.partial(
        pl.kernel,
        out_type=jax.ShapeDtypeStruct((NC, Npad, D), jnp.float32),
        mesh=mesh,
        compiler_params=pltpu.CompilerParams(use_tc_tiling_on_sc=False),
        scratch_types=[
            pltpu.VMEM((K, CHUNK), jnp.int32),
            pltpu.VMEM((K, CHUNK), jnp.int32),
            pltpu.VMEM((4, CHUNK, D), jnp.float32),
            pltpu.VMEM((NZT, D), jnp.float32),
            pltpu.VMEM((NTT, D), jnp.float32),
            pltpu.VMEM_SHARED((Npad, D), jnp.float32),
            pltpu.VMEM_SHARED((N, D), jnp.float32),
            [pltpu.SemaphoreType.DMA] * 4,
            [pltpu.SemaphoreType.DMA] * 4,
            pltpu.SemaphoreType.DMA,
            pltpu.SemaphoreType.DMA,
            pltpu.SemaphoreType.DMA,
        ],
    )
    def kfn(g_hbm, src_hbm, dst_hbm, zeros_hbm, out_hbm,
            src_v, dst_v, ring, zslab, tslab, acc, table,
            gsems, ssems, semA, semB, semC):
        cid = lax.axis_index("c")
        sid = lax.axis_index("s")
        wid = cid * NS + sid
        pltpu.async_copy(src_hbm.at[wid], src_v, semA)
        pltpu.async_copy(dst_hbm.at[wid], dst_v, semB)
        pltpu.async_copy(g_hbm.at[pl.ds(sid * NTT, NTT)], tslab, semC)
        pltpu.sync_copy(zeros_hbm, zslab)
        pltpu.sync_copy(zslab, acc.at[pl.ds(sid * NZT, NZT)])
        pltpu.make_async_copy(g_hbm.at[pl.ds(sid * NTT, NTT)], tslab,
                              semC).wait()
        pltpu.sync_copy(tslab, table.at[pl.ds(sid * NTT, NTT)])
        pltpu.make_async_copy(src_hbm.at[wid], src_v, semA).wait()
        pltpu.make_async_copy(dst_hbm.at[wid], dst_v, semB).wait()
        plsc.subcore_barrier()

        def gather(k, b):
            pltpu.async_copy(table.at[src_v.at[k]], ring.at[b], gsems[b])

        def wait_gather(b):
            pltpu.make_async_copy(table.at[src_v.at[0]], ring.at[b],
                                  gsems[b]).wait()

        def scatter(k, b):
            pltpu.async_copy(ring.at[b], acc.at[dst_v.at[k]], ssems[b],
                             add=True)

        def wait_scatter(b):
            pltpu.make_async_copy(ring.at[b], acc.at[dst_v.at[0]],
                                  ssems[b]).wait()

        gather(0, 0)
        gather(1, 1)
        wait_gather(0); scatter(0, 0); gather(2, 2)
        wait_gather(1); scatter(1, 1); gather(3, 3)

        def body(q, carry):
            k0 = 4 * q + 2
            for i in range(4):
                b = (2 + i) % 4
                b2 = (b + 2) % 4
                wait_gather(b)
                scatter(k0 + i, b)
                wait_scatter(b2)
                gather(k0 + i + 2, b2)
            return carry

        lax.fori_loop(0, (K - 4) // 4, body, 0)

        wait_gather(2); scatter(K - 2, 2)
        wait_gather(3); scatter(K - 1, 3)
        for b in range(4):
            wait_scatter(b)
        plsc.subcore_barrier()
        pltpu.sync_copy(acc.at[pl.ds(sid * NZT, NZT)],
                        out_hbm.at[cid].at[pl.ds(sid * NZT, NZT)])

    return kfn


def _deg_count(Npad, K, CHUNK):
    D = 8
    NZT = Npad // NS
    mesh = plsc.VectorSubcoreMesh(core_axis_name="c", subcore_axis_name="s")

    @functools.partial(
        pl.kernel,
        out_type=jax.ShapeDtypeStruct((NC, Npad, D), jnp.float32),
        mesh=mesh,
        compiler_params=pltpu.CompilerParams(use_tc_tiling_on_sc=False),
        scratch_types=[
            pltpu.VMEM((K, CHUNK), jnp.int32),
            pltpu.VMEM((CHUNK, D), jnp.float32),
            pltpu.VMEM((NZT, D), jnp.float32),
            pltpu.VMEM_SHARED((Npad, D), jnp.float32),
            pltpu.SemaphoreType.DMA,
            pltpu.SemaphoreType.DMA,
            pltpu.SemaphoreType.DMA,
        ],
    )
    def kfn(dst_hbm, ones_hbm, zeros_hbm, out_hbm, dst_v, ones_v, zslab, acc,
            ssem, semA, semB):
        cid = lax.axis_index("c")
        sid = lax.axis_index("s")
        wid = cid * NS + sid
        pltpu.async_copy(dst_hbm.at[wid], dst_v, semA)
        pltpu.async_copy(ones_hbm, ones_v, semB)
        pltpu.sync_copy(zeros_hbm, zslab)
        pltpu.sync_copy(zslab, acc.at[pl.ds(sid * NZT, NZT)])
        pltpu.make_async_copy(dst_hbm.at[wid], dst_v, semA).wait()
        pltpu.make_async_copy(ones_hbm, ones_v, semB).wait()
        plsc.subcore_barrier()

        def fire(k, carry):
            pltpu.async_copy(ones_v, acc.at[dst_v.at[k]], ssem, add=True)
            return carry

        lax.fori_loop(0, K, fire, 0)

        def drain(k, carry):
            pltpu.make_async_copy(ones_v, acc.at[dst_v.at[0]], ssem).wait()
            return carry

        lax.fori_loop(0, K, drain, 0)
        plsc.subcore_barrier()
        pltpu.sync_copy(acc.at[pl.ds(sid * NZT, NZT)],
                        out_hbm.at[cid].at[pl.ds(sid * NZT, NZT)])

    return kfn


def _tc_scale_mm(x, W, pdeg):
    N, D_in = x.shape
    D_out = W.shape[1]
    D_deg = pdeg.shape[2]
    BM = 2000 if N % 2000 == 0 else N

    def body(x_ref, w_ref, pd_ref, g_ref, dinv_ref):
        deg = pd_ref[0, :, 0:1] + pd_ref[1, :, 0:1]
        dinv = jnp.where(deg > 0, lax.rsqrt(deg), 0.0)
        h = jnp.dot(x_ref[...], w_ref[...], preferred_element_type=jnp.float32)
        g_ref[...] = h * dinv
        dinv_ref[...] = dinv

    return pl.pallas_call(
        body,
        grid=(N // BM,),
        in_specs=[
            pl.BlockSpec((BM, D_in), lambda i: (i, 0)),
            pl.BlockSpec((D_in, D_out), lambda i: (0, 0)),
            pl.BlockSpec((NC, BM, D_deg), lambda i: (0, i, 0)),
        ],
        out_specs=(pl.BlockSpec((BM, D_out), lambda i: (i, 0)),
                   pl.BlockSpec((BM, 1), lambda i: (i, 0))),
        out_shape=(jax.ShapeDtypeStruct((N, D_out), jnp.float32),
                   jax.ShapeDtypeStruct((N, 1), jnp.float32)),
    )(x, W, pdeg)


def _tc_mid(p1, dinv, b1, W2):
    N, D_out = dinv.shape[0], W2.shape[1]

    def body(p_ref, dinv_ref, b1_ref, w2_ref, g_ref):
        dinv = dinv_ref[...]
        s = p_ref[0, :N, :] + p_ref[1, :N, :]
        h = jnp.maximum(dinv * s + b1_ref[...], 0.0)
        g_ref[...] = dinv * jnp.dot(h, w2_ref[...],
                                    preferred_element_type=jnp.float32)

    return pl.pallas_call(
        body,
        out_shape=jax.ShapeDtypeStruct((N, D_out), jnp.float32),
    )(p1, dinv, b1, W2)


def _tc_final(p2, dinv, b2):
    N = dinv.shape[0]
    D_out = p2.shape[2]

    def body(p_ref, dinv_ref, b2_ref, o_ref):
        s = p_ref[0, :N, :] + p_ref[1, :N, :]
        o_ref[...] = dinv_ref[...] * s + b2_ref[...]

    return pl.pallas_call(
        body,
        out_shape=jax.ShapeDtypeStruct((N, D_out), jnp.float32),
    )(p2, dinv, b2)


def kernel(x, edge_index, W1, b1, W2, b2):
    N = x.shape[0]
    E = edge_index.shape[1]
    D_HID = W1.shape[1]
    D_OUT = W2.shape[1]
    Npad = (N // 128 + 1) * 128

    per_w = -(-E // NW)
    CHUNK = 0
    for c in range(128, 63, -1):
        if per_w * NW == E and per_w % c == 0 and (per_w // c) % 4 == 0:
            CHUNK = c
            break
    if CHUNK:
        K = per_w // CHUNK
        src_p = edge_index[0].reshape(NW, K, CHUNK)
        dst_p = edge_index[1].reshape(NW, K, CHUNK)
    else:
        CHUNK = 128
        K = -(-per_w // CHUNK)
        K += (-K) % 4
        pad = NW * K * CHUNK - E
        it = jnp.arange(pad, dtype=jnp.int32)
        src_p = jnp.concatenate([edge_index[0], it % N]).reshape(NW, K, CHUNK)
        dst_p = jnp.concatenate([edge_index[1], N + (it % (Npad - N))]
                                ).reshape(NW, K, CHUNK)

    NZT = Npad // NS
    ones8 = jnp.ones((CHUNK, 8), jnp.float32)
    zeros8 = jnp.zeros((NZT, 8), jnp.float32)
    zeros_h = jnp.zeros((NZT, D_HID), jnp.float32)
    zeros_o = jnp.zeros((NZT, D_OUT), jnp.float32)

    pdeg = _deg_count(Npad, K, CHUNK)(dst_p, ones8, zeros8)
    g1, dinv = _tc_scale_mm(x, W1, pdeg)
    p1 = _seg_sum(N, Npad, K, CHUNK, D_HID)(g1, src_p, dst_p, zeros_h)
    g2 = _tc_mid(p1, dinv, b1.reshape(1, -1), W2)
    p2 = _seg_sum(N, Npad, K, CHUNK, D_OUT)(g2, src_p, dst_p, zeros_o)
    out = _tc_final(p2, dinv, b2.reshape(1, -1))
    return (out, 0)


# speedup vs baseline: 1.0065x; 1.0065x over previous
import functools

import jax
import jax.numpy as jnp
from jax import lax
from jax.experimental import pallas as pl
from jax.experimental.pallas import tpu as pltpu
from jax.experimental.pallas import tpu_sc as plsc

NC = 2
NS = 16
NW = NC * NS


def _seg_sum(N, Npad, K, CHUNK, D):
    NZT = Npad // NS
    NTT = N // NS
    mesh = plsc.VectorSubcoreMesh(core_axis_name="c", subcore_axis_name="s")

    @functools.partial(
        pl.kernel,
        out_type=jax.ShapeDtypeStruct((NC, Npad, D), jnp.float32),
        mesh=mesh,
        compiler_params=pltpu.CompilerParams(use_tc_tiling_on_sc=False),
        scratch_types=[
            pltpu.VMEM((K, CHUNK), jnp.int32),
            pltpu.VMEM((K, CHUNK), jnp.int32),
            pltpu.VMEM((4, CHUNK, D), jnp.float32),
            pltpu.VMEM((NZT, D), jnp.float32),
            pltpu.VMEM((NTT, D), jnp.float32),
            pltpu.VMEM_SHARED((Npad, D), jnp.float32),
            pltpu.VMEM_SHARED((N, D), jnp.float32),
            [pltpu.SemaphoreType.DMA] * 4,
            [pltpu.SemaphoreType.DMA] * 4,
            pltpu.SemaphoreType.DMA,
            pltpu.SemaphoreType.DMA,
            pltpu.SemaphoreType.DMA,
        ],
    )
    def kfn(g_hbm, src_hbm, dst_hbm, zeros_hbm, out_hbm,
            src_v, dst_v, ring, zslab, tslab, acc, table,
            gsems, ssems, semA, semB, semC):
        cid = lax.axis_index("c")
        sid = lax.axis_index("s")
        wid = cid * NS + sid
        pltpu.async_copy(src_hbm.at[wid], src_v, semA)
        pltpu.async_copy(dst_hbm.at[wid], dst_v, semB)
        pltpu.async_copy(g_hbm.at[pl.ds(sid * NTT, NTT)], tslab, semC)
        pltpu.sync_copy(zeros_hbm, zslab)
        pltpu.sync_copy(zslab, acc.at[pl.ds(sid * NZT, NZT)])
        pltpu.make_async_copy(g_hbm.at[pl.ds(sid * NTT, NTT)], tslab,
                              semC).wait()
        pltpu.sync_copy(tslab, table.at[pl.ds(sid * NTT, NTT)])
        pltpu.make_async_copy(src_hbm.at[wid], src_v, semA).wait()
        pltpu.make_async_copy(dst_hbm.at[wid], dst_v, semB).wait()
        plsc.subcore_barrier()

        def gather(k, b):
            pltpu.async_copy(table.at[src_v.at[k]], ring.at[b], gsems[b])

        def wait_gather(b):
            pltpu.make_async_copy(table.at[src_v.at[0]], ring.at[b],
                                  gsems[b]).wait()

        def scatter(k, b):
            pltpu.async_copy(ring.at[b], acc.at[dst_v.at[k]], ssems[b],
                             add=True)

        def wait_scatter(b):
            pltpu.make_async_copy(ring.at[b], acc.at[dst_v.at[0]],
                                  ssems[b]).wait()

        gather(0, 0)
        gather(1, 1)
        wait_gather(0); scatter(0, 0); gather(2, 2)
        wait_gather(1); scatter(1, 1); gather(3, 3)

        def body(q, carry):
            k0 = 4 * q + 2
            for i in range(4):
                b = (2 + i) % 4
                b2 = (b + 2) % 4
                wait_gather(b)
                scatter(k0 + i, b)
                wait_scatter(b2)
                gather(k0 + i + 2, b2)
            return carry

        lax.fori_loop(0, (K - 4) // 4, body, 0)

        wait_gather(2); scatter(K - 2, 2)
        wait_gather(3); scatter(K - 1, 3)
        for b in range(4):
            wait_scatter(b)
        plsc.subcore_barrier()
        pltpu.sync_copy(acc.at[pl.ds(sid * NZT, NZT)],
                        out_hbm.at[cid].at[pl.ds(sid * NZT, NZT)])

    return kfn


def _deg_count(Npad, K, CHUNK):
    D = 4
    NZT = Npad // NS
    mesh = plsc.VectorSubcoreMesh(core_axis_name="c", subcore_axis_name="s")

    @functools.partial(
        pl.kernel,
        out_type=jax.ShapeDtypeStruct((NC, Npad, D), jnp.float32),
        mesh=mesh,
        compiler_params=pltpu.CompilerParams(use_tc_tiling_on_sc=False),
        scratch_types=[
            pltpu.VMEM((K, CHUNK), jnp.int32),
            pltpu.VMEM((CHUNK, D), jnp.float32),
            pltpu.VMEM((NZT, D), jnp.float32),
            pltpu.VMEM_SHARED((Npad, D), jnp.float32),
            pltpu.SemaphoreType.DMA,
            pltpu.SemaphoreType.DMA,
            pltpu.SemaphoreType.DMA,
        ],
    )
    def kfn(dst_hbm, ones_hbm, zeros_hbm, out_hbm, dst_v, ones_v, zslab, acc,
            ssem, semA, semB):
        cid = lax.axis_index("c")
        sid = lax.axis_index("s")
        wid = cid * NS + sid
        pltpu.async_copy(dst_hbm.at[wid], dst_v, semA)
        pltpu.async_copy(ones_hbm, ones_v, semB)
        pltpu.sync_copy(zeros_hbm, zslab)
        pltpu.sync_copy(zslab, acc.at[pl.ds(sid * NZT, NZT)])
        pltpu.make_async_copy(dst_hbm.at[wid], dst_v, semA).wait()
        pltpu.make_async_copy(ones_hbm, ones_v, semB).wait()
        plsc.subcore_barrier()

        def fire(k, carry):
            pltpu.async_copy(ones_v, acc.at[dst_v.at[k]], ssem, add=True)
            return carry

        lax.fori_loop(0, K, fire, 0)

        def drain(k, carry):
            pltpu.make_async_copy(ones_v, acc.at[dst_v.at[0]], ssem).wait()
            return carry

        lax.fori_loop(0, K, drain, 0)
        plsc.subcore_barrier()
        pltpu.sync_copy(acc.at[pl.ds(sid * NZT, NZT)],
                        out_hbm.at[cid].at[pl.ds(sid * NZT, NZT)])

    return kfn


def _tc_scale_mm(x, W, pdeg):
    N, D_out = x.shape[0], W.shape[1]

    def body(x_ref, w_ref, pd_ref, g_ref, dinv_ref):
        deg = pd_ref[0, :N, 0:1] + pd_ref[1, :N, 0:1]
        dinv = jnp.where(deg > 0, lax.rsqrt(deg), 0.0)
        h = jnp.dot(x_ref[...], w_ref[...], preferred_element_type=jnp.float32)
        g_ref[...] = h * dinv
        dinv_ref[...] = dinv

    return pl.pallas_call(
        body,
        out_shape=(jax.ShapeDtypeStruct((N, D_out), jnp.float32),
                   jax.ShapeDtypeStruct((N, 1), jnp.float32)),
    )(x, W, pdeg)


def _tc_mid(p1, dinv, b1, W2):
    N, D_out = dinv.shape[0], W2.shape[1]

    def body(p_ref, dinv_ref, b1_ref, w2_ref, g_ref):
        dinv = dinv_ref[...]
        s = p_ref[0, :N, :] + p_ref[1, :N, :]
        h = jnp.maximum(dinv * s + b1_ref[...], 0.0)
        g_ref[...] = dinv * jnp.dot(h, w2_ref[...],
                                    preferred_element_type=jnp.float32)

    return pl.pallas_call(
        body,
        out_shape=jax.ShapeDtypeStruct((N, D_out), jnp.float32),
    )(p1, dinv, b1, W2)


def _tc_final(p2, dinv, b2):
    N = dinv.shape[0]
    D_out = p2.shape[2]

    def body(p_ref, dinv_ref, b2_ref, o_ref):
        s = p_ref[0, :N, :] + p_ref[1, :N, :]
        o_ref[...] = dinv_ref[...] * s + b2_ref[...]

    return pl.pallas_call(
        body,
        out_shape=jax.ShapeDtypeStruct((N, D_out), jnp.float32),
    )(p2, dinv, b2)


def kernel(x, edge_index, W1, b1, W2, b2):
    N = x.shape[0]
    E = edge_index.shape[1]
    D_HID = W1.shape[1]
    D_OUT = W2.shape[1]
    Npad = (N // 128 + 1) * 128

    per_w = -(-E // NW)
    CHUNK = 0
    for c in range(128, 63, -1):
        if per_w * NW == E and per_w % c == 0 and (per_w // c) % 4 == 0:
            CHUNK = c
            break
    if CHUNK:
        K = per_w // CHUNK
        src_p = edge_index[0].reshape(NW, K, CHUNK)
        dst_p = edge_index[1].reshape(NW, K, CHUNK)
    else:
        CHUNK = 128
        K = -(-per_w // CHUNK)
        K += (-K) % 4
        pad = NW * K * CHUNK - E
        it = jnp.arange(pad, dtype=jnp.int32)
        src_p = jnp.concatenate([edge_index[0], it % N]).reshape(NW, K, CHUNK)
        dst_p = jnp.concatenate([edge_index[1], N + (it % (Npad - N))]
                                ).reshape(NW, K, CHUNK)

    NZT = Npad // NS
    ones8 = jnp.ones((CHUNK, 4), jnp.float32)
    zeros8 = jnp.zeros((NZT, 4), jnp.float32)
    zeros_h = jnp.zeros((NZT, D_HID), jnp.float32)
    zeros_o = jnp.zeros((NZT, D_OUT), jnp.float32)

    pdeg = _deg_count(Npad, K, CHUNK)(dst_p, ones8, zeros8)
    g1, dinv = _tc_scale_mm(x, W1, pdeg)
    p1 = _seg_sum(N, Npad, K, CHUNK, D_HID)(g1, src_p, dst_p, zeros_h)
    g2 = _tc_mid(p1, dinv, b1.reshape(1, -1), W2)
    p2 = _seg_sum(N, Npad, K, CHUNK, D_OUT)(g2, src_p, dst_p, zeros_o)
    out = _tc_final(p2, dinv, b2.reshape(1, -1))
    return (out, 0)
